# Initial kernel scaffold; baseline (speedup 1.0000x reference)
import functools

import jax
import jax.numpy as jnp
from jax import lax
from jax.experimental import pallas as pl
from jax.experimental.pallas import tpu as pltpu
from jax.experimental.pallas import tpu_sc as plsc

DIM = 64
B = 16384
L = 20
N = B * L
NW = 32
PER_W = N // NW
IDXROW = 128
CHUNK = 512
ROWS = CHUNK // IDXROW
NITER = PER_W // CHUNK

_mesh = plsc.VectorSubcoreMesh(core_axis_name="c", subcore_axis_name="s")


@functools.partial(
    pl.kernel,
    mesh=_mesh,
    out_type=jax.ShapeDtypeStruct((N, DIM), jnp.float32),
    scratch_types=[
        pltpu.VMEM((ROWS, IDXROW), jnp.int32),
        pltpu.VMEM((CHUNK, DIM), jnp.float32),
        pltpu.SemaphoreType.DMA,
    ],
)
def _emb_lookup(x_hbm, table_hbm, out_hbm, idx_v, rows_v, sem):
    wid = lax.axis_index("s") * 2 + lax.axis_index("c")
    row_base = wid * (PER_W // IDXROW)

    def body(g, carry):
        r0 = row_base + g * ROWS
        pltpu.sync_copy(x_hbm.at[pl.ds(r0, ROWS)], idx_v)
        copies = [
            pltpu.async_copy(
                table_hbm.at[idx_v.at[j]],
                rows_v.at[pl.ds(j * IDXROW, IDXROW)],
                sem,
            )
            for j in range(ROWS)
        ]
        for cp in copies:
            cp.wait()
        pltpu.sync_copy(rows_v, out_hbm.at[pl.ds(r0 * IDXROW, CHUNK)])
        return carry

    lax.fori_loop(0, NITER, body, 0)


def kernel(x, table):
    xf = x.astype(jnp.int32).reshape(N // IDXROW, IDXROW)
    out = _emb_lookup(xf, table)
    return out.reshape(B, L, DIM)


# speedup vs baseline: 3.9652x; 3.9652x over previous
import functools

import jax
import jax.numpy as jnp
from jax import lax
from jax.experimental import pallas as pl
from jax.experimental.pallas import tpu as pltpu
from jax.experimental.pallas import tpu_sc as plsc

DIM = 64
B = 16384
L = 20
N = B * L
NW = 32
PER_W = N // NW
IDXROW = 128
CHUNK = 512
ROWS = CHUNK // IDXROW
NITER = PER_W // CHUNK

_mesh = plsc.VectorSubcoreMesh(core_axis_name="c", subcore_axis_name="s")


@functools.partial(
    pl.kernel,
    mesh=_mesh,
    out_type=jax.ShapeDtypeStruct((N, DIM), jnp.float32),
    scratch_types=[
        pltpu.VMEM((ROWS, IDXROW), jnp.int32),
        pltpu.VMEM((CHUNK, DIM), jnp.float32),
        pltpu.SemaphoreType.DMA,
    ],
    compiler_params=pltpu.CompilerParams(use_tc_tiling_on_sc=False),
)
def _emb_lookup(x_hbm, table_hbm, out_hbm, idx_v, rows_v, sem):
    wid = lax.axis_index("s") * 2 + lax.axis_index("c")
    row_base = wid * (PER_W // IDXROW)

    def body(g, carry):
        r0 = row_base + g * ROWS
        pltpu.sync_copy(x_hbm.at[pl.ds(r0, ROWS)], idx_v)
        copies = [
            pltpu.async_copy(
                table_hbm.at[idx_v.at[j]],
                rows_v.at[pl.ds(j * IDXROW, IDXROW)],
                sem,
            )
            for j in range(ROWS)
        ]
        for cp in copies:
            cp.wait()
        pltpu.sync_copy(rows_v, out_hbm.at[pl.ds(r0 * IDXROW, CHUNK)])
        return carry

    lax.fori_loop(0, NITER, body, 0)


def kernel(x, table):
    xf = x.astype(jnp.int32).reshape(N // IDXROW, IDXROW)
    out = _emb_lookup(xf, table)
    return out.reshape(B, L, DIM)


# speedup vs baseline: 4.1320x; 1.0421x over previous
import functools

import jax
import jax.numpy as jnp
from jax import lax
from jax.experimental import pallas as pl
from jax.experimental.pallas import tpu as pltpu
from jax.experimental.pallas import tpu_sc as plsc

DIM = 64
B = 16384
L = 20
N = B * L
NW = 32
PER_W = N // NW
IDXROW = 128
CHUNK = 512
ROWS = CHUNK // IDXROW
NITER = PER_W // CHUNK
NROW_W = PER_W // IDXROW

_mesh = plsc.VectorSubcoreMesh(core_axis_name="c", subcore_axis_name="s")


@functools.partial(
    pl.kernel,
    mesh=_mesh,
    out_type=jax.ShapeDtypeStruct((N, DIM), jnp.float32),
    scratch_types=[
        pltpu.VMEM((NROW_W, IDXROW), jnp.int32),
        pltpu.VMEM((CHUNK, DIM), jnp.float32),
        pltpu.VMEM((CHUNK, DIM), jnp.float32),
        pltpu.SemaphoreType.DMA,
        pltpu.SemaphoreType.DMA,
        pltpu.SemaphoreType.DMA,
        pltpu.SemaphoreType.DMA,
    ],
    compiler_params=pltpu.CompilerParams(use_tc_tiling_on_sc=False),
)
def _emb_lookup(x_hbm, table_hbm, out_hbm, idx_v, rows0, rows1,
                gsem0, gsem1, ssem0, ssem1):
    wid = lax.axis_index("s") * 2 + lax.axis_index("c")
    row_base = wid * NROW_W
    out_base = wid * PER_W

    pltpu.sync_copy(x_hbm.at[pl.ds(row_base, NROW_W)], idx_v)

    def fire(g, rows_buf, sem):
        for j in range(ROWS):
            pltpu.async_copy(
                table_hbm.at[idx_v.at[g * ROWS + j]],
                rows_buf.at[pl.ds(j * IDXROW, IDXROW)],
                sem,
            )

    def wait_gather(g, rows_buf, sem):
        for j in range(ROWS):
            pltpu.make_async_copy(
                table_hbm.at[idx_v.at[g * ROWS + j]],
                rows_buf.at[pl.ds(j * IDXROW, IDXROW)],
                sem,
            ).wait()

    def store(g, rows_buf, sem):
        return pltpu.async_copy(
            rows_buf, out_hbm.at[pl.ds(out_base + g * CHUNK, CHUNK)], sem)

    def wait_store(g, rows_buf, sem):
        pltpu.make_async_copy(
            rows_buf, out_hbm.at[pl.ds(out_base + g * CHUNK, CHUNK)], sem,
        ).wait()

    fire(0, rows0, gsem0)

    def body(h, carry):
        g0 = 2 * h
        fire(g0 + 1, rows1, gsem1)
        wait_gather(g0, rows0, gsem0)
        store(g0, rows0, ssem0)
        wait_gather(g0 + 1, rows1, gsem1)
        store(g0 + 1, rows1, ssem1)
        wait_store(g0, rows0, ssem0)
        fire(g0 + 2, rows0, gsem0)
        wait_store(g0 + 1, rows1, ssem1)
        return carry

    lax.fori_loop(0, NITER // 2 - 1, body, 0)

    glast = NITER - 2
    fire(glast + 1, rows1, gsem1)
    wait_gather(glast, rows0, gsem0)
    pltpu.sync_copy(rows0, out_hbm.at[pl.ds(out_base + glast * CHUNK, CHUNK)])
    wait_gather(glast + 1, rows1, gsem1)
    pltpu.sync_copy(rows1,
                    out_hbm.at[pl.ds(out_base + (glast + 1) * CHUNK, CHUNK)])


def kernel(x, table):
    xf = x.astype(jnp.int32).reshape(N // IDXROW, IDXROW)
    out = _emb_lookup(xf, table)
    return out.reshape(B, L, DIM)
